# Initial kernel scaffold; baseline (speedup 1.0000x reference)
import jax
import jax.numpy as jnp
from jax.experimental import pallas as pl


def kernel(x, edge_index, Wl0, Wr0, b0, Wl1, Wr1, b1, Wl2, Wr2, b2, Wp, bp):
    raise NotImplementedError("write your pallas kernel here")


# speedup vs baseline: 3.2878x; 3.2878x over previous
import functools

import jax
import jax.numpy as jnp
from jax import lax
from jax.experimental import pallas as pl
from jax.experimental.pallas import tpu as pltpu
from jax.experimental.pallas import tpu_sc as plsc

N = 10000
E = 320000
F = 128
CH = 128
NC = 2
NS = 16
NW = NC * NS
CPW = 80
SLAB = 40
EPAD = NW * CPW * CH - E
ROWS_PER_TILE = 632
NACC = NS * ROWS_PER_TILE
CNTW = 128

_ZERO_SPANS = ((0, 128), (128, 128), (256, 128), (384, 128), (512, 120))


def _sc_agg_body(h_hbm, src_hbm, dst_hbm, sum_hbm,
                 src_v, dst_v, rows0, rows1, sem0, sem1, acc_sh):
    c = lax.axis_index("c")
    s = lax.axis_index("s")
    wid = c * NS + s

    zv = jnp.zeros((16,), jnp.float32)

    def zero_row(r, _):
        for j in range(F // 16):
            rows0[r, pl.ds(j * 16, 16)] = zv
        return 0

    lax.fori_loop(0, CH, zero_row, 0)

    base_row = s * ROWS_PER_TILE
    for off, nrows in _ZERO_SPANS:
        pltpu.sync_copy(rows0.at[pl.ds(0, nrows)],
                        acc_sh.at[pl.ds(base_row + off, nrows)])
    plsc.subcore_barrier()

    rows = (rows0, rows1)
    sems = (sem0, sem1)

    def gather_start(k, b):
        pltpu.async_copy(h_hbm.at[src_v.at[k]], rows[b], sems[b])

    def gather_wait(k, b):
        pltpu.make_async_copy(h_hbm.at[src_v.at[k]], rows[b], sems[b]).wait()

    def scatter(k, b):
        pltpu.sync_copy(rows[b], acc_sh.at[dst_v.at[k]], add=True)

    for slab in range(CPW // SLAB):
        pltpu.sync_copy(src_hbm.at[wid, pl.ds(slab * SLAB, SLAB)], src_v)
        pltpu.sync_copy(dst_hbm.at[wid, pl.ds(slab * SLAB, SLAB)], dst_v)

        gather_start(0, 0)

        def pair_body(i, _):
            k0 = 2 * i
            gather_start(k0 + 1, 1)
            gather_wait(k0, 0)
            scatter(k0, 0)

            @pl.when(k0 + 2 < SLAB)
            def _next():
                gather_start(k0 + 2, 0)

            gather_wait(k0 + 1, 1)
            scatter(k0 + 1, 1)
            return 0

        lax.fori_loop(0, SLAB // 2, pair_body, 0)

    plsc.subcore_barrier()

    pltpu.sync_copy(acc_sh.at[pl.ds(base_row, ROWS_PER_TILE)],
                    sum_hbm.at[c, pl.ds(base_row, ROWS_PER_TILE)])


def _sc_cnt_body(dst_hbm, cnt_hbm, dst_v, ones_v, cz_v, cnt_sh):
    c = lax.axis_index("c")
    s = lax.axis_index("s")
    wid = c * NS + s

    zv = jnp.zeros((16,), jnp.float32)
    ov = jnp.ones((16,), jnp.float32)

    def fill_row(r, _):
        for j in range(CNTW // 16):
            ones_v[r, pl.ds(j * 16, 16)] = ov
            cz_v[r, pl.ds(j * 16, 16)] = zv
        return 0

    lax.fori_loop(0, CH, fill_row, 0)

    base_row = s * ROWS_PER_TILE
    for off, nrows in _ZERO_SPANS:
        pltpu.sync_copy(cz_v.at[pl.ds(0, nrows)],
                        cnt_sh.at[pl.ds(base_row + off, nrows)])
    plsc.subcore_barrier()

    pltpu.sync_copy(dst_hbm.at[wid], dst_v)

    def chunk_body(k, _):
        pltpu.sync_copy(ones_v, cnt_sh.at[dst_v.at[k]], add=True)
        return 0

    lax.fori_loop(0, CPW, chunk_body, 0)
    plsc.subcore_barrier()

    pltpu.sync_copy(cnt_sh.at[pl.ds(base_row, ROWS_PER_TILE)],
                    cnt_hbm.at[c, pl.ds(base_row, ROWS_PER_TILE)])


def _sc_mesh():
    return plsc.VectorSubcoreMesh(core_axis_name="c", subcore_axis_name="s",
                                  num_cores=NC, num_subcores=NS)


@functools.cache
def _make_sc_agg():
    return pl.kernel(
        _sc_agg_body,
        out_type=jax.ShapeDtypeStruct((NC, NACC, F), jnp.float32),
        mesh=_sc_mesh(),
        scratch_types=[
            pltpu.VMEM((SLAB, CH), jnp.int32),
            pltpu.VMEM((SLAB, CH), jnp.int32),
            pltpu.VMEM((CH, F), jnp.float32),
            pltpu.VMEM((CH, F), jnp.float32),
            pltpu.SemaphoreType.DMA,
            pltpu.SemaphoreType.DMA,
            pltpu.VMEM_SHARED((NACC, F), jnp.float32),
        ],
    )


@functools.cache
def _make_sc_cnt():
    return pl.kernel(
        _sc_cnt_body,
        out_type=jax.ShapeDtypeStruct((NC, NACC, CNTW), jnp.float32),
        mesh=_sc_mesh(),
        scratch_types=[
            pltpu.VMEM((CPW, CH), jnp.int32),
            pltpu.VMEM((CH, CNTW), jnp.float32),
            pltpu.VMEM((CH, CNTW), jnp.float32),
            pltpu.VMEM_SHARED((NACC, CNTW), jnp.float32),
        ],
    )


def _dense_body(sum_ref, cnt_ref, h_ref, wl_ref, wr_ref, b_ref, out_ref):
    agg = sum_ref[0] + sum_ref[1]
    cnt = cnt_ref[0, :, 0:1] + cnt_ref[1, :, 0:1]
    mean = agg * (1.0 / jnp.maximum(cnt, 1.0))
    acc = jnp.dot(mean, wl_ref[...], preferred_element_type=jnp.float32)
    acc = acc + jnp.dot(h_ref[...], wr_ref[...], preferred_element_type=jnp.float32)
    out_ref[...] = jnp.maximum(acc + b_ref[...], 0.0)


def _final_body(sum_ref, cnt_ref, h2_ref, wl_ref, wr_ref, b_ref,
                h1_ref, p1_ref, p2_ref, p3_ref, bp_ref, out_ref):
    agg = sum_ref[0] + sum_ref[1]
    cnt = cnt_ref[0, :, 0:1] + cnt_ref[1, :, 0:1]
    mean = agg * (1.0 / jnp.maximum(cnt, 1.0))
    acc = jnp.dot(mean, wl_ref[...], preferred_element_type=jnp.float32)
    acc = acc + jnp.dot(h2_ref[...], wr_ref[...], preferred_element_type=jnp.float32)
    h3 = jnp.maximum(acc + b_ref[...], 0.0)
    out = jnp.dot(h1_ref[...], p1_ref[...], preferred_element_type=jnp.float32)
    out = out + jnp.dot(h2_ref[...], p2_ref[...], preferred_element_type=jnp.float32)
    out = out + jnp.dot(h3, p3_ref[...], preferred_element_type=jnp.float32)
    out_ref[...] = out + bp_ref[...]


_BLK = ROWS_PER_TILE
_GRID = NACC // _BLK

_row_spec = pl.BlockSpec((_BLK, F), lambda i: (i, 0))
_sum_spec = pl.BlockSpec((NC, _BLK, F), lambda i: (0, i, 0))
_cnt_spec = pl.BlockSpec((NC, _BLK, CNTW), lambda i: (0, i, 0))
_w_spec = pl.BlockSpec((F, F), lambda i: (0, 0))
_b_spec = pl.BlockSpec((1, F), lambda i: (0, 0))


def _dense(sum2, cnt2, h, wlT, wrT, b):
    return pl.pallas_call(
        _dense_body,
        grid=(_GRID,),
        in_specs=[_sum_spec, _cnt_spec, _row_spec, _w_spec, _w_spec, _b_spec],
        out_specs=_row_spec,
        out_shape=jax.ShapeDtypeStruct((NACC, F), jnp.float32),
    )(sum2, cnt2, h, wlT, wrT, b)


def _final(sum2, cnt2, h2, wlT, wrT, b, h1, p1, p2, p3, bp):
    return pl.pallas_call(
        _final_body,
        grid=(_GRID,),
        in_specs=[_sum_spec, _cnt_spec, _row_spec, _w_spec, _w_spec, _b_spec,
                  _row_spec, _w_spec, _w_spec, _w_spec, _b_spec],
        out_specs=_row_spec,
        out_shape=jax.ShapeDtypeStruct((NACC, F), jnp.float32),
    )(sum2, cnt2, h2, wlT, wrT, b, h1, p1, p2, p3, bp)


def kernel(x, edge_index, Wl0, Wr0, b0, Wl1, Wr1, b1, Wl2, Wr2, b2, Wp, bp):
    sc_agg = _make_sc_agg()
    sc_cnt = _make_sc_cnt()

    src_p = jnp.concatenate(
        [edge_index[0], jnp.zeros((EPAD,), jnp.int32)]).reshape(NW, CPW, CH)
    dst_p = jnp.concatenate(
        [edge_index[1], jnp.full((EPAD,), N, jnp.int32)]).reshape(NW, CPW, CH)
    xp = jnp.pad(x, ((0, NACC - N), (0, 0)))

    cnt = sc_cnt(dst_p)
    sum0 = sc_agg(xp, src_p, dst_p)
    h1 = _dense(sum0, cnt, xp, Wl0.T, Wr0.T, b0.reshape(1, F))
    sum1 = sc_agg(h1, src_p, dst_p)
    h2 = _dense(sum1, cnt, h1, Wl1.T, Wr1.T, b1.reshape(1, F))
    sum2 = sc_agg(h2, src_p, dst_p)
    out = _final(sum2, cnt, h2, Wl2.T, Wr2.T, b2.reshape(1, F),
                 h1, Wp[:, :F].T, Wp[:, F:2 * F].T, Wp[:, 2 * F:].T,
                 bp.reshape(1, F))
    return out[:N]


# speedup vs baseline: 7.8563x; 2.3895x over previous
import functools

import jax
import jax.numpy as jnp
from jax import lax
from jax.experimental import pallas as pl
from jax.experimental.pallas import tpu as pltpu
from jax.experimental.pallas import tpu_sc as plsc

N = 10000
E = 320000
F = 128
CH = 128
NC = 2
NS = 16
NW = NC * NS
CPW = 79
EPAD = NW * CPW * CH - E
ROWS_PER_TILE = 632
NACC = NS * ROWS_PER_TILE
NPAD = NACC - N
CNTW = 128

_ZERO_SPANS = ((0, 128), (128, 128), (256, 128), (384, 128), (512, 120))


def _sc_agg_body(h_hbm, src_hbm, dst_hbm, sum_hbm, src_v, dst_v, rows_v, acc_sh):
    c = lax.axis_index("c")
    s = lax.axis_index("s")
    wid = c * NS + s

    zv = jnp.zeros((16,), jnp.float32)

    def zero_row(r, _):
        for j in range(F // 16):
            rows_v[r, pl.ds(j * 16, 16)] = zv
        return 0

    lax.fori_loop(0, CH, zero_row, 0)

    base_row = s * ROWS_PER_TILE
    for off, nrows in _ZERO_SPANS:
        pltpu.sync_copy(rows_v.at[pl.ds(0, nrows)],
                        acc_sh.at[pl.ds(base_row + off, nrows)])
    plsc.subcore_barrier()

    pltpu.sync_copy(src_hbm.at[wid], src_v)
    pltpu.sync_copy(dst_hbm.at[wid], dst_v)

    def chunk_body(k, _):
        pltpu.sync_copy(h_hbm.at[src_v.at[k]], rows_v)
        pltpu.sync_copy(rows_v, acc_sh.at[dst_v.at[k]], add=True)
        return 0

    lax.fori_loop(0, CPW, chunk_body, 0)
    plsc.subcore_barrier()

    pltpu.sync_copy(acc_sh.at[pl.ds(base_row, ROWS_PER_TILE)],
                    sum_hbm.at[c, pl.ds(base_row, ROWS_PER_TILE)])


def _sc_cnt_body(dst_hbm, cnt_hbm, dst_v, ones_v, cz_v, cnt_sh):
    c = lax.axis_index("c")
    s = lax.axis_index("s")
    wid = c * NS + s

    zv = jnp.zeros((16,), jnp.float32)
    ov = jnp.ones((16,), jnp.float32)

    def fill_row(r, _):
        for j in range(CNTW // 16):
            ones_v[r, pl.ds(j * 16, 16)] = ov
            cz_v[r, pl.ds(j * 16, 16)] = zv
        return 0

    lax.fori_loop(0, CH, fill_row, 0)

    base_row = s * ROWS_PER_TILE
    for off, nrows in _ZERO_SPANS:
        pltpu.sync_copy(cz_v.at[pl.ds(0, nrows)],
                        cnt_sh.at[pl.ds(base_row + off, nrows)])
    plsc.subcore_barrier()

    pltpu.sync_copy(dst_hbm.at[wid], dst_v)

    def chunk_body(k, _):
        pltpu.sync_copy(ones_v, cnt_sh.at[dst_v.at[k]], add=True)
        return 0

    lax.fori_loop(0, CPW, chunk_body, 0)
    plsc.subcore_barrier()

    pltpu.sync_copy(cnt_sh.at[pl.ds(base_row, ROWS_PER_TILE)],
                    cnt_hbm.at[c, pl.ds(base_row, ROWS_PER_TILE)])


def _sc_mesh():
    return plsc.VectorSubcoreMesh(core_axis_name="c", subcore_axis_name="s",
                                  num_cores=NC, num_subcores=NS)


@functools.cache
def _make_sc_agg():
    return pl.kernel(
        _sc_agg_body,
        out_type=jax.ShapeDtypeStruct((NC, NACC, F), jnp.float32),
        mesh=_sc_mesh(),
        scratch_types=[
            pltpu.VMEM((CPW, CH), jnp.int32),
            pltpu.VMEM((CPW, CH), jnp.int32),
            pltpu.VMEM((CH, F), jnp.float32),
            pltpu.VMEM_SHARED((NACC, F), jnp.float32),
        ],
    )


@functools.cache
def _make_sc_cnt():
    return pl.kernel(
        _sc_cnt_body,
        out_type=jax.ShapeDtypeStruct((NC, NACC, CNTW), jnp.float32),
        mesh=_sc_mesh(),
        scratch_types=[
            pltpu.VMEM((CPW, CH), jnp.int32),
            pltpu.VMEM((CH, CNTW), jnp.float32),
            pltpu.VMEM((CH, CNTW), jnp.float32),
            pltpu.VMEM_SHARED((NACC, CNTW), jnp.float32),
        ],
    )


def _dense_body(sum_ref, cnt_ref, h_ref, wl_ref, wr_ref, b_ref, out_ref):
    agg = sum_ref[0] + sum_ref[1]
    cnt = cnt_ref[0, :, 0:1] + cnt_ref[1, :, 0:1]
    mean = agg * (1.0 / jnp.maximum(cnt, 1.0))
    acc = jnp.dot(mean, wl_ref[...], preferred_element_type=jnp.float32)
    acc = acc + jnp.dot(h_ref[...], wr_ref[...], preferred_element_type=jnp.float32)
    out_ref[...] = jnp.maximum(acc + b_ref[...], 0.0)


def _final_body(sum_ref, cnt_ref, h2_ref, wl_ref, wr_ref, b_ref,
                h1_ref, p1_ref, p2_ref, p3_ref, bp_ref, out_ref):
    agg = sum_ref[0] + sum_ref[1]
    cnt = cnt_ref[0, :, 0:1] + cnt_ref[1, :, 0:1]
    mean = agg * (1.0 / jnp.maximum(cnt, 1.0))
    acc = jnp.dot(mean, wl_ref[...], preferred_element_type=jnp.float32)
    acc = acc + jnp.dot(h2_ref[...], wr_ref[...], preferred_element_type=jnp.float32)
    h3 = jnp.maximum(acc + b_ref[...], 0.0)
    out = jnp.dot(h1_ref[...], p1_ref[...], preferred_element_type=jnp.float32)
    out = out + jnp.dot(h2_ref[...], p2_ref[...], preferred_element_type=jnp.float32)
    out = out + jnp.dot(h3, p3_ref[...], preferred_element_type=jnp.float32)
    out_ref[...] = out + bp_ref[...]


_BLK = ROWS_PER_TILE
_GRID = NACC // _BLK

_row_spec = pl.BlockSpec((_BLK, F), lambda i: (i, 0))
_sum_spec = pl.BlockSpec((NC, _BLK, F), lambda i: (0, i, 0))
_cnt_spec = pl.BlockSpec((NC, _BLK, CNTW), lambda i: (0, i, 0))
_w_spec = pl.BlockSpec((F, F), lambda i: (0, 0))
_b_spec = pl.BlockSpec((1, F), lambda i: (0, 0))


def _dense(sum2, cnt2, h, wlT, wrT, b):
    return pl.pallas_call(
        _dense_body,
        grid=(_GRID,),
        in_specs=[_sum_spec, _cnt_spec, _row_spec, _w_spec, _w_spec, _b_spec],
        out_specs=_row_spec,
        out_shape=jax.ShapeDtypeStruct((NACC, F), jnp.float32),
    )(sum2, cnt2, h, wlT, wrT, b)


def _final(sum2, cnt2, h2, wlT, wrT, b, h1, p1, p2, p3, bp):
    return pl.pallas_call(
        _final_body,
        grid=(_GRID,),
        in_specs=[_sum_spec, _cnt_spec, _row_spec, _w_spec, _w_spec, _b_spec,
                  _row_spec, _w_spec, _w_spec, _w_spec, _b_spec],
        out_specs=_row_spec,
        out_shape=jax.ShapeDtypeStruct((NACC, F), jnp.float32),
    )(sum2, cnt2, h2, wlT, wrT, b, h1, p1, p2, p3, bp)


def kernel(x, edge_index, Wl0, Wr0, b0, Wl1, Wr1, b1, Wl2, Wr2, b2, Wp, bp):
    sc_agg = _make_sc_agg()
    sc_cnt = _make_sc_cnt()

    pad_ids = jnp.arange(EPAD, dtype=jnp.int32)
    src_p = jnp.concatenate(
        [edge_index[0], pad_ids % N]).reshape(NW, CPW, CH)
    dst_p = jnp.concatenate(
        [edge_index[1], N + pad_ids % NPAD]).reshape(NW, CPW, CH)
    xp = jnp.pad(x, ((0, NACC - N), (0, 0)))

    cnt = sc_cnt(dst_p)
    sum0 = sc_agg(xp, src_p, dst_p)
    h1 = _dense(sum0, cnt, xp, Wl0.T, Wr0.T, b0.reshape(1, F))
    sum1 = sc_agg(h1, src_p, dst_p)
    h2 = _dense(sum1, cnt, h1, Wl1.T, Wr1.T, b1.reshape(1, F))
    sum2 = sc_agg(h2, src_p, dst_p)
    out = _final(sum2, cnt, h2, Wl2.T, Wr2.T, b2.reshape(1, F),
                 h1, Wp[:, :F].T, Wp[:, F:2 * F].T, Wp[:, 2 * F:].T,
                 bp.reshape(1, F))
    return out[:N]


# speedup vs baseline: 9.7265x; 1.2380x over previous
import functools

import jax
import jax.numpy as jnp
from jax import lax
from jax.experimental import pallas as pl
from jax.experimental.pallas import tpu as pltpu
from jax.experimental.pallas import tpu_sc as plsc

N = 10000
E = 320000
F = 128
CH = 128
NC = 2
NS = 16
NW = NC * NS
CPW = 80
SLAB = 40
EPAD = NW * CPW * CH - E
ROWS_PER_TILE = 632
NACC = NS * ROWS_PER_TILE
NPAD = NACC - N
CNTW = 128

_ZERO_SPANS = ((0, 128), (128, 128), (256, 128), (384, 128), (512, 120))


def _sc_agg_body(h_hbm, src_hbm, dst_hbm, sum_hbm,
                 src_v, dst_v, rows_a, rows_b, sga, sgb, ssa, ssb, acc_sh):
    c = lax.axis_index("c")
    s = lax.axis_index("s")
    wid = c * NS + s

    zv = jnp.zeros((16,), jnp.float32)

    def zero_row(r, _):
        for j in range(F // 16):
            rows_a[r, pl.ds(j * 16, 16)] = zv
        return 0

    lax.fori_loop(0, CH, zero_row, 0)

    base_row = s * ROWS_PER_TILE
    for off, nrows in _ZERO_SPANS:
        pltpu.sync_copy(rows_a.at[pl.ds(0, nrows)],
                        acc_sh.at[pl.ds(base_row + off, nrows)])
    plsc.subcore_barrier()

    def g_start(k, buf, sem):
        pltpu.async_copy(h_hbm.at[src_v.at[k]], buf, sem)

    def g_wait(k, buf, sem):
        pltpu.make_async_copy(h_hbm.at[src_v.at[k]], buf, sem).wait()

    def s_start(k, buf, sem):
        pltpu.async_copy(buf, acc_sh.at[dst_v.at[k]], sem, add=True)

    def s_wait(k, buf, sem):
        pltpu.make_async_copy(buf, acc_sh.at[dst_v.at[k]], sem).wait()

    for slab in range(CPW // SLAB):
        base = slab * SLAB
        pltpu.sync_copy(src_hbm.at[wid, pl.ds(base, SLAB)], src_v)
        pltpu.sync_copy(dst_hbm.at[wid, pl.ds(base, SLAB)], dst_v)

        g_start(0, rows_a, sga)
        g_wait(0, rows_a, sga)
        s_start(0, rows_a, ssa)
        g_start(1, rows_b, sgb)
        g_wait(1, rows_b, sgb)
        s_start(1, rows_b, ssb)
        s_wait(0, rows_a, ssa)
        g_start(2, rows_a, sga)

        def pair_body(i, _):
            k0 = 2 * i
            g_wait(k0, rows_a, sga)
            s_start(k0, rows_a, ssa)
            s_wait(k0 - 1, rows_b, ssb)
            g_start(k0 + 1, rows_b, sgb)
            g_wait(k0 + 1, rows_b, sgb)
            s_start(k0 + 1, rows_b, ssb)
            s_wait(k0, rows_a, ssa)
            g_start(k0 + 2, rows_a, sga)
            return 0

        lax.fori_loop(1, SLAB // 2 - 1, pair_body, 0)

        g_wait(SLAB - 2, rows_a, sga)
        s_start(SLAB - 2, rows_a, ssa)
        s_wait(SLAB - 3, rows_b, ssb)
        g_start(SLAB - 1, rows_b, sgb)
        g_wait(SLAB - 1, rows_b, sgb)
        s_start(SLAB - 1, rows_b, ssb)
        s_wait(SLAB - 2, rows_a, ssa)
        s_wait(SLAB - 1, rows_b, ssb)

    plsc.subcore_barrier()

    pltpu.sync_copy(acc_sh.at[pl.ds(base_row, ROWS_PER_TILE)],
                    sum_hbm.at[c, pl.ds(base_row, ROWS_PER_TILE)])


def _sc_cnt_body(dst_hbm, cnt_hbm, dst_v, ones_v, cz_v, cnt_sh):
    c = lax.axis_index("c")
    s = lax.axis_index("s")
    wid = c * NS + s

    zv = jnp.zeros((16,), jnp.float32)
    ov = jnp.ones((16,), jnp.float32)

    def fill_row(r, _):
        for j in range(CNTW // 16):
            ones_v[r, pl.ds(j * 16, 16)] = ov
            cz_v[r, pl.ds(j * 16, 16)] = zv
        return 0

    lax.fori_loop(0, CH, fill_row, 0)

    base_row = s * ROWS_PER_TILE
    for off, nrows in _ZERO_SPANS:
        pltpu.sync_copy(cz_v.at[pl.ds(0, nrows)],
                        cnt_sh.at[pl.ds(base_row + off, nrows)])
    plsc.subcore_barrier()

    pltpu.sync_copy(dst_hbm.at[wid], dst_v)

    def chunk_body(k, _):
        pltpu.sync_copy(ones_v, cnt_sh.at[dst_v.at[k]], add=True)
        return 0

    lax.fori_loop(0, CPW, chunk_body, 0)
    plsc.subcore_barrier()

    pltpu.sync_copy(cnt_sh.at[pl.ds(base_row, ROWS_PER_TILE)],
                    cnt_hbm.at[c, pl.ds(base_row, ROWS_PER_TILE)])


def _sc_mesh():
    return plsc.VectorSubcoreMesh(core_axis_name="c", subcore_axis_name="s",
                                  num_cores=NC, num_subcores=NS)


@functools.cache
def _make_sc_agg():
    return pl.kernel(
        _sc_agg_body,
        out_type=jax.ShapeDtypeStruct((NC, NACC, F), jnp.float32),
        mesh=_sc_mesh(),
        scratch_types=[
            pltpu.VMEM((SLAB, CH), jnp.int32),
            pltpu.VMEM((SLAB, CH), jnp.int32),
            pltpu.VMEM((CH, F), jnp.float32),
            pltpu.VMEM((CH, F), jnp.float32),
            pltpu.SemaphoreType.DMA,
            pltpu.SemaphoreType.DMA,
            pltpu.SemaphoreType.DMA,
            pltpu.SemaphoreType.DMA,
            pltpu.VMEM_SHARED((NACC, F), jnp.float32),
        ],
    )


@functools.cache
def _make_sc_cnt():
    return pl.kernel(
        _sc_cnt_body,
        out_type=jax.ShapeDtypeStruct((NC, NACC, CNTW), jnp.float32),
        mesh=_sc_mesh(),
        scratch_types=[
            pltpu.VMEM((CPW, CH), jnp.int32),
            pltpu.VMEM((CH, CNTW), jnp.float32),
            pltpu.VMEM((CH, CNTW), jnp.float32),
            pltpu.VMEM_SHARED((NACC, CNTW), jnp.float32),
        ],
    )


def _dense_body(sum_ref, cnt_ref, h_ref, wl_ref, wr_ref, b_ref, out_ref):
    agg = sum_ref[0] + sum_ref[1]
    cnt = cnt_ref[0, :, 0:1] + cnt_ref[1, :, 0:1]
    mean = agg * (1.0 / jnp.maximum(cnt, 1.0))
    acc = jnp.dot(mean, wl_ref[...], preferred_element_type=jnp.float32)
    acc = acc + jnp.dot(h_ref[...], wr_ref[...], preferred_element_type=jnp.float32)
    out_ref[...] = jnp.maximum(acc + b_ref[...], 0.0)


def _final_body(sum_ref, cnt_ref, h2_ref, wl_ref, wr_ref, b_ref,
                h1_ref, p1_ref, p2_ref, p3_ref, bp_ref, out_ref):
    agg = sum_ref[0] + sum_ref[1]
    cnt = cnt_ref[0, :, 0:1] + cnt_ref[1, :, 0:1]
    mean = agg * (1.0 / jnp.maximum(cnt, 1.0))
    acc = jnp.dot(mean, wl_ref[...], preferred_element_type=jnp.float32)
    acc = acc + jnp.dot(h2_ref[...], wr_ref[...], preferred_element_type=jnp.float32)
    h3 = jnp.maximum(acc + b_ref[...], 0.0)
    out = jnp.dot(h1_ref[...], p1_ref[...], preferred_element_type=jnp.float32)
    out = out + jnp.dot(h2_ref[...], p2_ref[...], preferred_element_type=jnp.float32)
    out = out + jnp.dot(h3, p3_ref[...], preferred_element_type=jnp.float32)
    out_ref[...] = out + bp_ref[...]


_BLK = ROWS_PER_TILE
_GRID = NACC // _BLK

_row_spec = pl.BlockSpec((_BLK, F), lambda i: (i, 0))
_sum_spec = pl.BlockSpec((NC, _BLK, F), lambda i: (0, i, 0))
_cnt_spec = pl.BlockSpec((NC, _BLK, CNTW), lambda i: (0, i, 0))
_w_spec = pl.BlockSpec((F, F), lambda i: (0, 0))
_b_spec = pl.BlockSpec((1, F), lambda i: (0, 0))


def _dense(sum2, cnt2, h, wlT, wrT, b):
    return pl.pallas_call(
        _dense_body,
        grid=(_GRID,),
        in_specs=[_sum_spec, _cnt_spec, _row_spec, _w_spec, _w_spec, _b_spec],
        out_specs=_row_spec,
        out_shape=jax.ShapeDtypeStruct((NACC, F), jnp.float32),
    )(sum2, cnt2, h, wlT, wrT, b)


def _final(sum2, cnt2, h2, wlT, wrT, b, h1, p1, p2, p3, bp):
    return pl.pallas_call(
        _final_body,
        grid=(_GRID,),
        in_specs=[_sum_spec, _cnt_spec, _row_spec, _w_spec, _w_spec, _b_spec,
                  _row_spec, _w_spec, _w_spec, _w_spec, _b_spec],
        out_specs=_row_spec,
        out_shape=jax.ShapeDtypeStruct((NACC, F), jnp.float32),
    )(sum2, cnt2, h2, wlT, wrT, b, h1, p1, p2, p3, bp)


def kernel(x, edge_index, Wl0, Wr0, b0, Wl1, Wr1, b1, Wl2, Wr2, b2, Wp, bp):
    sc_agg = _make_sc_agg()
    sc_cnt = _make_sc_cnt()

    pad_ids = jnp.arange(EPAD, dtype=jnp.int32)
    src_p = jnp.concatenate(
        [edge_index[0], pad_ids % N]).reshape(NW, CPW, CH)
    dst_p = jnp.concatenate(
        [edge_index[1], N + pad_ids % NPAD]).reshape(NW, CPW, CH)
    xp = jnp.pad(x, ((0, NACC - N), (0, 0)))

    cnt = sc_cnt(dst_p)
    sum0 = sc_agg(xp, src_p, dst_p)
    h1 = _dense(sum0, cnt, xp, Wl0.T, Wr0.T, b0.reshape(1, F))
    sum1 = sc_agg(h1, src_p, dst_p)
    h2 = _dense(sum1, cnt, h1, Wl1.T, Wr1.T, b1.reshape(1, F))
    sum2 = sc_agg(h2, src_p, dst_p)
    out = _final(sum2, cnt, h2, Wl2.T, Wr2.T, b2.reshape(1, F),
                 h1, Wp[:, :F].T, Wp[:, F:2 * F].T, Wp[:, 2 * F:].T,
                 bp.reshape(1, F))
    return out[:N]


# speedup vs baseline: 9.7360x; 1.0010x over previous
import functools

import jax
import jax.numpy as jnp
from jax import lax
from jax.experimental import pallas as pl
from jax.experimental.pallas import tpu as pltpu
from jax.experimental.pallas import tpu_sc as plsc

N = 10000
E = 320000
F = 128
CH = 128
NC = 2
NS = 16
NW = NC * NS
CPW = 80
SLAB = 40
EPAD = NW * CPW * CH - E
ROWS_PER_TILE = 632
NACC = NS * ROWS_PER_TILE
NPAD = NACC - N
CNTW = 128

_ZERO_SPANS = ((0, 128), (128, 128), (256, 128), (384, 128), (512, 120))


def _sc_agg_body(h_hbm, src_hbm, dst_hbm, sum_hbm,
                 src_v, dst_v, rows_a, rows_b, sga, sgb, ssa, ssb, acc_sh):
    c = lax.axis_index("c")
    s = lax.axis_index("s")
    wid = c * NS + s

    zv = jnp.zeros((16,), jnp.float32)

    def zero_row(r, _):
        for j in range(F // 16):
            rows_a[r, pl.ds(j * 16, 16)] = zv
        return 0

    lax.fori_loop(0, CH, zero_row, 0)

    base_row = s * ROWS_PER_TILE
    for off, nrows in _ZERO_SPANS:
        pltpu.sync_copy(rows_a.at[pl.ds(0, nrows)],
                        acc_sh.at[pl.ds(base_row + off, nrows)])
    plsc.subcore_barrier()

    def g_start(k, buf, sem):
        pltpu.async_copy(h_hbm.at[src_v.at[k]], buf, sem)

    def g_wait(k, buf, sem):
        pltpu.make_async_copy(h_hbm.at[src_v.at[k]], buf, sem).wait()

    def s_start(k, buf, sem):
        pltpu.async_copy(buf, acc_sh.at[dst_v.at[k]], sem, add=True)

    def s_wait(k, buf, sem):
        pltpu.make_async_copy(buf, acc_sh.at[dst_v.at[k]], sem).wait()

    for slab in range(CPW // SLAB):
        base = slab * SLAB
        pltpu.sync_copy(src_hbm.at[wid, pl.ds(base, SLAB)], src_v)
        pltpu.sync_copy(dst_hbm.at[wid, pl.ds(base, SLAB)], dst_v)

        g_start(0, rows_a, sga)
        g_wait(0, rows_a, sga)
        s_start(0, rows_a, ssa)
        g_start(1, rows_b, sgb)
        g_wait(1, rows_b, sgb)
        s_start(1, rows_b, ssb)
        s_wait(0, rows_a, ssa)
        g_start(2, rows_a, sga)

        def pair_body(i, _):
            k0 = 2 * i
            g_wait(k0, rows_a, sga)
            s_start(k0, rows_a, ssa)
            s_wait(k0 - 1, rows_b, ssb)
            g_start(k0 + 1, rows_b, sgb)
            g_wait(k0 + 1, rows_b, sgb)
            s_start(k0 + 1, rows_b, ssb)
            s_wait(k0, rows_a, ssa)
            g_start(k0 + 2, rows_a, sga)
            return 0

        lax.fori_loop(1, SLAB // 2 - 1, pair_body, 0)

        g_wait(SLAB - 2, rows_a, sga)
        s_start(SLAB - 2, rows_a, ssa)
        s_wait(SLAB - 3, rows_b, ssb)
        g_start(SLAB - 1, rows_b, sgb)
        g_wait(SLAB - 1, rows_b, sgb)
        s_start(SLAB - 1, rows_b, ssb)
        s_wait(SLAB - 2, rows_a, ssa)
        s_wait(SLAB - 1, rows_b, ssb)

    plsc.subcore_barrier()

    pltpu.sync_copy(acc_sh.at[pl.ds(base_row, ROWS_PER_TILE)],
                    sum_hbm.at[c, pl.ds(base_row, ROWS_PER_TILE)])


def _sc_cnt_body(dst_hbm, cnt_hbm, dst_v, ones_v, cz_v, sem, cnt_sh):
    c = lax.axis_index("c")
    s = lax.axis_index("s")
    wid = c * NS + s

    zv = jnp.zeros((16,), jnp.float32)
    ov = jnp.ones((16,), jnp.float32)

    def fill_row(r, _):
        for j in range(CNTW // 16):
            ones_v[r, pl.ds(j * 16, 16)] = ov
            cz_v[r, pl.ds(j * 16, 16)] = zv
        return 0

    lax.fori_loop(0, CH, fill_row, 0)

    base_row = s * ROWS_PER_TILE
    for off, nrows in _ZERO_SPANS:
        pltpu.sync_copy(cz_v.at[pl.ds(0, nrows)],
                        cnt_sh.at[pl.ds(base_row + off, nrows)])
    plsc.subcore_barrier()

    pltpu.sync_copy(dst_hbm.at[wid], dst_v)

    def chunk_body(k, _):
        pltpu.async_copy(ones_v, cnt_sh.at[dst_v.at[k]], sem, add=True)
        return 0

    lax.fori_loop(0, CPW, chunk_body, 0)

    def drain_body(k, _):
        pltpu.make_async_copy(ones_v, cnt_sh.at[dst_v.at[k]], sem).wait()
        return 0

    lax.fori_loop(0, CPW, drain_body, 0)
    plsc.subcore_barrier()

    pltpu.sync_copy(cnt_sh.at[pl.ds(base_row, ROWS_PER_TILE)],
                    cnt_hbm.at[c, pl.ds(base_row, ROWS_PER_TILE)])


def _sc_mesh():
    return plsc.VectorSubcoreMesh(core_axis_name="c", subcore_axis_name="s",
                                  num_cores=NC, num_subcores=NS)


@functools.cache
def _make_sc_agg():
    return pl.kernel(
        _sc_agg_body,
        out_type=jax.ShapeDtypeStruct((NC, NACC, F), jnp.float32),
        mesh=_sc_mesh(),
        scratch_types=[
            pltpu.VMEM((SLAB, CH), jnp.int32),
            pltpu.VMEM((SLAB, CH), jnp.int32),
            pltpu.VMEM((CH, F), jnp.float32),
            pltpu.VMEM((CH, F), jnp.float32),
            pltpu.SemaphoreType.DMA,
            pltpu.SemaphoreType.DMA,
            pltpu.SemaphoreType.DMA,
            pltpu.SemaphoreType.DMA,
            pltpu.VMEM_SHARED((NACC, F), jnp.float32),
        ],
    )


@functools.cache
def _make_sc_cnt():
    return pl.kernel(
        _sc_cnt_body,
        out_type=jax.ShapeDtypeStruct((NC, NACC, CNTW), jnp.float32),
        mesh=_sc_mesh(),
        scratch_types=[
            pltpu.VMEM((CPW, CH), jnp.int32),
            pltpu.VMEM((CH, CNTW), jnp.float32),
            pltpu.VMEM((CH, CNTW), jnp.float32),
            pltpu.SemaphoreType.DMA,
            pltpu.VMEM_SHARED((NACC, CNTW), jnp.float32),
        ],
    )


def _dense_body(sum_ref, cnt_ref, h_ref, wl_ref, wr_ref, b_ref, out_ref):
    agg = sum_ref[0] + sum_ref[1]
    cnt = cnt_ref[0, :, 0:1] + cnt_ref[1, :, 0:1]
    mean = agg * (1.0 / jnp.maximum(cnt, 1.0))
    acc = jnp.dot(mean, wl_ref[...], preferred_element_type=jnp.float32)
    acc = acc + jnp.dot(h_ref[...], wr_ref[...], preferred_element_type=jnp.float32)
    out_ref[...] = jnp.maximum(acc + b_ref[...], 0.0)


def _final_body(sum_ref, cnt_ref, h2_ref, wl_ref, wr_ref, b_ref,
                h1_ref, p1_ref, p2_ref, p3_ref, bp_ref, out_ref):
    agg = sum_ref[0] + sum_ref[1]
    cnt = cnt_ref[0, :, 0:1] + cnt_ref[1, :, 0:1]
    mean = agg * (1.0 / jnp.maximum(cnt, 1.0))
    acc = jnp.dot(mean, wl_ref[...], preferred_element_type=jnp.float32)
    acc = acc + jnp.dot(h2_ref[...], wr_ref[...], preferred_element_type=jnp.float32)
    h3 = jnp.maximum(acc + b_ref[...], 0.0)
    out = jnp.dot(h1_ref[...], p1_ref[...], preferred_element_type=jnp.float32)
    out = out + jnp.dot(h2_ref[...], p2_ref[...], preferred_element_type=jnp.float32)
    out = out + jnp.dot(h3, p3_ref[...], preferred_element_type=jnp.float32)
    out_ref[...] = out + bp_ref[...]


_BLK = ROWS_PER_TILE
_GRID = NACC // _BLK

_row_spec = pl.BlockSpec((_BLK, F), lambda i: (i, 0))
_sum_spec = pl.BlockSpec((NC, _BLK, F), lambda i: (0, i, 0))
_cnt_spec = pl.BlockSpec((NC, _BLK, CNTW), lambda i: (0, i, 0))
_w_spec = pl.BlockSpec((F, F), lambda i: (0, 0))
_b_spec = pl.BlockSpec((1, F), lambda i: (0, 0))


def _dense(sum2, cnt2, h, wlT, wrT, b):
    return pl.pallas_call(
        _dense_body,
        grid=(_GRID,),
        in_specs=[_sum_spec, _cnt_spec, _row_spec, _w_spec, _w_spec, _b_spec],
        out_specs=_row_spec,
        out_shape=jax.ShapeDtypeStruct((NACC, F), jnp.float32),
    )(sum2, cnt2, h, wlT, wrT, b)


def _final(sum2, cnt2, h2, wlT, wrT, b, h1, p1, p2, p3, bp):
    return pl.pallas_call(
        _final_body,
        grid=(_GRID,),
        in_specs=[_sum_spec, _cnt_spec, _row_spec, _w_spec, _w_spec, _b_spec,
                  _row_spec, _w_spec, _w_spec, _w_spec, _b_spec],
        out_specs=_row_spec,
        out_shape=jax.ShapeDtypeStruct((NACC, F), jnp.float32),
    )(sum2, cnt2, h2, wlT, wrT, b, h1, p1, p2, p3, bp)


def kernel(x, edge_index, Wl0, Wr0, b0, Wl1, Wr1, b1, Wl2, Wr2, b2, Wp, bp):
    sc_agg = _make_sc_agg()
    sc_cnt = _make_sc_cnt()

    pad_ids = jnp.arange(EPAD, dtype=jnp.int32)
    src_p = jnp.concatenate(
        [edge_index[0], pad_ids % N]).reshape(NW, CPW, CH)
    dst_p = jnp.concatenate(
        [edge_index[1], N + pad_ids % NPAD]).reshape(NW, CPW, CH)
    xp = jnp.pad(x, ((0, NACC - N), (0, 0)))

    cnt = sc_cnt(dst_p)
    sum0 = sc_agg(xp, src_p, dst_p)
    h1 = _dense(sum0, cnt, xp, Wl0.T, Wr0.T, b0.reshape(1, F))
    sum1 = sc_agg(h1, src_p, dst_p)
    h2 = _dense(sum1, cnt, h1, Wl1.T, Wr1.T, b1.reshape(1, F))
    sum2 = sc_agg(h2, src_p, dst_p)
    out = _final(sum2, cnt, h2, Wl2.T, Wr2.T, b2.reshape(1, F),
                 h1, Wp[:, :F].T, Wp[:, F:2 * F].T, Wp[:, 2 * F:].T,
                 bp.reshape(1, F))
    return out[:N]


# speedup vs baseline: 9.7918x; 1.0057x over previous
import functools

import jax
import jax.numpy as jnp
from jax import lax
from jax.experimental import pallas as pl
from jax.experimental.pallas import tpu as pltpu
from jax.experimental.pallas import tpu_sc as plsc

N = 10000
E = 320000
F = 128
CH = 128
NC = 2
NS = 16
NW = NC * NS
CPW = 80
SLAB = 40
EPAD = NW * CPW * CH - E
ROWS_PER_TILE = 632
NACC = NS * ROWS_PER_TILE
NPAD = NACC - N
CNTW = 128

_ZERO_SPANS = ((0, 128), (128, 128), (256, 128), (384, 128), (512, 120))


def _sc_agg_body(with_cnt, h_hbm, src_hbm, dst_hbm, *refs):
    if with_cnt:
        (sum_hbm, cnt_hbm, src_v, dst_v, rows_a, rows_b,
         sga, sgb, ssa, ssb, acc_sh) = refs
    else:
        (sum_hbm, src_v, dst_v, rows_a, rows_b,
         sga, sgb, ssa, ssb, acc_sh) = refs
        cnt_hbm = None

    c = lax.axis_index("c")
    s = lax.axis_index("s")
    wid = c * NS + s

    zv = jnp.zeros((16,), jnp.float32)

    def zero_row(r, _):
        for j in range(F // 16):
            rows_a[r, pl.ds(j * 16, 16)] = zv
        return 0

    lax.fori_loop(0, CH, zero_row, 0)

    if with_cnt:
        ov = jnp.ones((16,), jnp.float32)

        def ones_row(r, _):
            for j in range(F // 16):
                rows_b[r, pl.ds(j * 16, 16)] = ov
            return 0

        lax.fori_loop(0, CH, ones_row, 0)

    base_row = s * ROWS_PER_TILE

    def zero_acc():
        for off, nrows in _ZERO_SPANS:
            pltpu.sync_copy(rows_a.at[pl.ds(0, nrows)],
                            acc_sh.at[pl.ds(base_row + off, nrows)])

    zero_acc()
    plsc.subcore_barrier()

    if with_cnt:
        for slab in range(CPW // SLAB):
            pltpu.sync_copy(dst_hbm.at[wid, pl.ds(slab * SLAB, SLAB)], dst_v)

            def cnt_start(k, _):
                pltpu.async_copy(rows_b, acc_sh.at[dst_v.at[k]], ssa, add=True)
                return 0

            lax.fori_loop(0, SLAB, cnt_start, 0)

            def cnt_drain(k, _):
                pltpu.make_async_copy(rows_b, acc_sh.at[dst_v.at[k]], ssa).wait()
                return 0

            lax.fori_loop(0, SLAB, cnt_drain, 0)

        plsc.subcore_barrier()
        pltpu.sync_copy(acc_sh.at[pl.ds(base_row, ROWS_PER_TILE)],
                        cnt_hbm.at[c, pl.ds(base_row, ROWS_PER_TILE)])
        zero_acc()
        plsc.subcore_barrier()

    def g_start(k, buf, sem):
        pltpu.async_copy(h_hbm.at[src_v.at[k]], buf, sem)

    def g_wait(k, buf, sem):
        pltpu.make_async_copy(h_hbm.at[src_v.at[k]], buf, sem).wait()

    def s_start(k, buf, sem):
        pltpu.async_copy(buf, acc_sh.at[dst_v.at[k]], sem, add=True)

    def s_wait(k, buf, sem):
        pltpu.make_async_copy(buf, acc_sh.at[dst_v.at[k]], sem).wait()

    for slab in range(CPW // SLAB):
        base = slab * SLAB
        pltpu.sync_copy(src_hbm.at[wid, pl.ds(base, SLAB)], src_v)
        pltpu.sync_copy(dst_hbm.at[wid, pl.ds(base, SLAB)], dst_v)

        g_start(0, rows_a, sga)
        g_wait(0, rows_a, sga)
        s_start(0, rows_a, ssa)
        g_start(1, rows_b, sgb)
        g_wait(1, rows_b, sgb)
        s_start(1, rows_b, ssb)
        s_wait(0, rows_a, ssa)
        g_start(2, rows_a, sga)

        def pair_body(i, _):
            k0 = 2 * i
            g_wait(k0, rows_a, sga)
            s_start(k0, rows_a, ssa)
            s_wait(k0 - 1, rows_b, ssb)
            g_start(k0 + 1, rows_b, sgb)
            g_wait(k0 + 1, rows_b, sgb)
            s_start(k0 + 1, rows_b, ssb)
            s_wait(k0, rows_a, ssa)
            g_start(k0 + 2, rows_a, sga)
            return 0

        lax.fori_loop(1, SLAB // 2 - 1, pair_body, 0)

        g_wait(SLAB - 2, rows_a, sga)
        s_start(SLAB - 2, rows_a, ssa)
        s_wait(SLAB - 3, rows_b, ssb)
        g_start(SLAB - 1, rows_b, sgb)
        g_wait(SLAB - 1, rows_b, sgb)
        s_start(SLAB - 1, rows_b, ssb)
        s_wait(SLAB - 2, rows_a, ssa)
        s_wait(SLAB - 1, rows_b, ssb)

    plsc.subcore_barrier()

    pltpu.sync_copy(acc_sh.at[pl.ds(base_row, ROWS_PER_TILE)],
                    sum_hbm.at[c, pl.ds(base_row, ROWS_PER_TILE)])


def _sc_mesh():
    return plsc.VectorSubcoreMesh(core_axis_name="c", subcore_axis_name="s",
                                  num_cores=NC, num_subcores=NS)


@functools.cache
def _make_sc_agg(with_cnt):
    out_type = [jax.ShapeDtypeStruct((NC, NACC, F), jnp.float32)]
    if with_cnt:
        out_type = out_type + [jax.ShapeDtypeStruct((NC, NACC, CNTW), jnp.float32)]
    return pl.kernel(
        functools.partial(_sc_agg_body, with_cnt),
        out_type=out_type,
        mesh=_sc_mesh(),
        scratch_types=[
            pltpu.VMEM((SLAB, CH), jnp.int32),
            pltpu.VMEM((SLAB, CH), jnp.int32),
            pltpu.VMEM((CH, F), jnp.float32),
            pltpu.VMEM((CH, F), jnp.float32),
            pltpu.SemaphoreType.DMA,
            pltpu.SemaphoreType.DMA,
            pltpu.SemaphoreType.DMA,
            pltpu.SemaphoreType.DMA,
            pltpu.VMEM_SHARED((NACC, F), jnp.float32),
        ],
    )


def _dense_body(sum_ref, cnt_ref, h_ref, wl_ref, wr_ref, b_ref, out_ref):
    agg = sum_ref[0] + sum_ref[1]
    cnt = cnt_ref[0, :, 0:1] + cnt_ref[1, :, 0:1]
    mean = agg * (1.0 / jnp.maximum(cnt, 1.0))
    acc = jnp.dot(mean, wl_ref[...], preferred_element_type=jnp.float32)
    acc = acc + jnp.dot(h_ref[...], wr_ref[...], preferred_element_type=jnp.float32)
    out_ref[...] = jnp.maximum(acc + b_ref[...], 0.0)


def _final_body(sum_ref, cnt_ref, h2_ref, wl_ref, wr_ref, b_ref,
                h1_ref, p1_ref, p2_ref, p3_ref, bp_ref, out_ref):
    agg = sum_ref[0] + sum_ref[1]
    cnt = cnt_ref[0, :, 0:1] + cnt_ref[1, :, 0:1]
    mean = agg * (1.0 / jnp.maximum(cnt, 1.0))
    acc = jnp.dot(mean, wl_ref[...], preferred_element_type=jnp.float32)
    acc = acc + jnp.dot(h2_ref[...], wr_ref[...], preferred_element_type=jnp.float32)
    h3 = jnp.maximum(acc + b_ref[...], 0.0)
    out = jnp.dot(h1_ref[...], p1_ref[...], preferred_element_type=jnp.float32)
    out = out + jnp.dot(h2_ref[...], p2_ref[...], preferred_element_type=jnp.float32)
    out = out + jnp.dot(h3, p3_ref[...], preferred_element_type=jnp.float32)
    out_ref[...] = out + bp_ref[...]


_BLK = ROWS_PER_TILE
_GRID = NACC // _BLK

_row_spec = pl.BlockSpec((_BLK, F), lambda i: (i, 0))
_sum_spec = pl.BlockSpec((NC, _BLK, F), lambda i: (0, i, 0))
_cnt_spec = pl.BlockSpec((NC, _BLK, CNTW), lambda i: (0, i, 0))
_w_spec = pl.BlockSpec((F, F), lambda i: (0, 0))
_b_spec = pl.BlockSpec((1, F), lambda i: (0, 0))


def _dense(sum2, cnt2, h, wlT, wrT, b):
    return pl.pallas_call(
        _dense_body,
        grid=(_GRID,),
        in_specs=[_sum_spec, _cnt_spec, _row_spec, _w_spec, _w_spec, _b_spec],
        out_specs=_row_spec,
        out_shape=jax.ShapeDtypeStruct((NACC, F), jnp.float32),
    )(sum2, cnt2, h, wlT, wrT, b)


def _final(sum2, cnt2, h2, wlT, wrT, b, h1, p1, p2, p3, bp):
    return pl.pallas_call(
        _final_body,
        grid=(_GRID,),
        in_specs=[_sum_spec, _cnt_spec, _row_spec, _w_spec, _w_spec, _b_spec,
                  _row_spec, _w_spec, _w_spec, _w_spec, _b_spec],
        out_specs=_row_spec,
        out_shape=jax.ShapeDtypeStruct((NACC, F), jnp.float32),
    )(sum2, cnt2, h2, wlT, wrT, b, h1, p1, p2, p3, bp)


def kernel(x, edge_index, Wl0, Wr0, b0, Wl1, Wr1, b1, Wl2, Wr2, b2, Wp, bp):
    sc_agg_first = _make_sc_agg(True)
    sc_agg = _make_sc_agg(False)

    pad_ids = jnp.arange(EPAD, dtype=jnp.int32)
    src_p = jnp.concatenate(
        [edge_index[0], pad_ids % N]).reshape(NW, CPW, CH)
    dst_p = jnp.concatenate(
        [edge_index[1], N + pad_ids % NPAD]).reshape(NW, CPW, CH)
    xp = jnp.pad(x, ((0, NACC - N), (0, 0)))

    sum0, cnt = sc_agg_first(xp, src_p, dst_p)
    h1 = _dense(sum0, cnt, xp, Wl0.T, Wr0.T, b0.reshape(1, F))
    sum1, = sc_agg(h1, src_p, dst_p)
    h2 = _dense(sum1, cnt, h1, Wl1.T, Wr1.T, b1.reshape(1, F))
    sum2, = sc_agg(h2, src_p, dst_p)
    out = _final(sum2, cnt, h2, Wl2.T, Wr2.T, b2.reshape(1, F),
                 h1, Wp[:, :F].T, Wp[:, F:2 * F].T, Wp[:, 2 * F:].T,
                 bp.reshape(1, F))
    return out[:N]
